# baseline (speedup 1.0000x reference)
import functools

import jax
import jax.numpy as jnp
from jax import lax
from jax.experimental import pallas as pl
from jax.experimental.pallas import tpu as pltpu
from jax.experimental.pallas import tpu_sc as plsc

_NC = 2
_NS = 16
_NW = _NC * _NS
_LANES = 16

_D = 32
_K = 20
_C = 64
_IDX_CHUNK = 128


def _sc_scores_body(c_hbm, x_hbm, n_hbm, cemb_hbm, xemb_hbm,
                    pos_hbm, negs_hbm,
                    cidx, xidx, nidx, crows, xrows, nrows, posb, negb, sem,
                    *, pb):
    wid = lax.axis_index("s") * _NC + lax.axis_index("c")
    base = wid * pb

    pltpu.sync_copy(c_hbm.at[pl.ds(base, pb)], cidx)
    pltpu.sync_copy(x_hbm.at[pl.ds(base, pb)], xidx)
    pltpu.sync_copy(n_hbm.at[pl.ds(base * _K, pb * _K)], nidx)

    lane = lax.iota(jnp.int32, _LANES)

    def subchunk(c, _):
        co = c * _C
        descs = []
        descs.append(pltpu.async_copy(
            cemb_hbm.at[cidx.at[pl.ds(co, _C)]], crows, sem))
        descs.append(pltpu.async_copy(
            xemb_hbm.at[xidx.at[pl.ds(co, _C)]], xrows, sem))
        nrows2d = nrows
        n_rows_total = _C * _K
        for j in range(n_rows_total // _IDX_CHUNK):
            descs.append(pltpu.async_copy(
                xemb_hbm.at[nidx.at[pl.ds(co * _K + j * _IDX_CHUNK, _IDX_CHUNK)]],
                nrows2d.at[pl.ds(j * _IDX_CHUNK, _IDX_CHUNK)], sem))
        for dsc in descs:
            dsc.wait()

        def group(g, _):
            rows = g * _LANES + lane
            nbase = rows * _K
            rbase = rows * _D
            nfbase = nbase * _D

            def dstep(d, accs):
                dcol = jnp.full((_LANES,), d, jnp.int32)
                cvec = plsc.load_gather(crows, [rows, dcol])
                xvec = plsc.load_gather(xrows, [rows, dcol])
                out = [accs[0] + cvec * xvec]
                for k in range(_K):
                    nvec = plsc.load_gather(nrows, [nbase + k, dcol])
                    out.append(accs[k + 1] + cvec * nvec)
                return tuple(out)

            zero = jnp.zeros((_LANES,), jnp.float32)
            accs = lax.fori_loop(0, _D, dstep, (zero,) * (_K + 1))
            plsc.store_scatter(posb, [rows], accs[0])
            for k in range(_K):
                plsc.store_scatter(negb, [nbase + k], accs[k + 1])
            return 0

        lax.fori_loop(0, _C // _LANES, group, 0)

        pltpu.sync_copy(posb, pos_hbm.at[pl.ds(base + co, _C)])
        pltpu.sync_copy(negb, negs_hbm.at[pl.ds((base + co) * _K, _C * _K)])
        return 0

    lax.fori_loop(0, pb // _C, subchunk, 0)


def _sc_scores(center, context, neg_flat, center_emb, context_emb):
    b = center.shape[0]
    pb = b // _NW
    mesh = plsc.VectorSubcoreMesh(core_axis_name="c", subcore_axis_name="s")
    fn = pl.kernel(
        functools.partial(_sc_scores_body, pb=pb),
        out_type=(
            jax.ShapeDtypeStruct((b,), jnp.float32),
            jax.ShapeDtypeStruct((b * _K,), jnp.float32),
        ),
        mesh=mesh,
        scratch_types=[
            pltpu.VMEM((pb,), jnp.int32),
            pltpu.VMEM((pb,), jnp.int32),
            pltpu.VMEM((pb * _K,), jnp.int32),
            pltpu.VMEM((_C, _D), jnp.float32),
            pltpu.VMEM((_C, _D), jnp.float32),
            pltpu.VMEM((_C * _K, _D), jnp.float32),
            pltpu.VMEM((_C,), jnp.float32),
            pltpu.VMEM((_C * _K,), jnp.float32),
            pltpu.SemaphoreType.DMA,
        ],
        compiler_params=pltpu.CompilerParams(needs_layout_passes=False, use_tc_tiling_on_sc=False),
        name="w2v_sc_scores",
    )
    return fn(center, context, neg_flat, center_emb, context_emb)


def _loss_body(pos_ref, neg_ref, out_ref, *, b, k):
    p = pos_ref[...]
    n = neg_ref[...]
    sp = 1.0 / (1.0 + jnp.exp(-p))
    sn = 1.0 / (1.0 + jnp.exp(-n))
    lp = jnp.log(sp + 1e-9)
    ln = jnp.log(1.0 - sn + 1e-9)
    loss = -(jnp.sum(lp) / b) - (jnp.sum(ln) / (b * k))
    out_ref[...] = jnp.full((1, 1), loss, jnp.float32)


def _tc_loss(pos2d, neg2d, b, k):
    fn = pl.pallas_call(
        functools.partial(_loss_body, b=b, k=k),
        out_shape=jax.ShapeDtypeStruct((1, 1), jnp.float32),
    )
    return fn(pos2d, neg2d)


def kernel(center, context, negative_samples, center_emb, context_emb):
    b = center.shape[0]
    k = negative_samples.shape[1]
    neg_flat = negative_samples.reshape(b * k)
    pos, negs = _sc_scores(center, context, neg_flat, center_emb, context_emb)
    pos2d = pos.reshape(b // 128, 128)
    neg2d = negs.reshape((b * k) // 128, 128)
    loss = _tc_loss(pos2d, neg2d, b, k)
    return loss[0, 0]


# speedup vs baseline: 1.0566x; 1.0566x over previous
import functools

import jax
import jax.numpy as jnp
from jax import lax
from jax.experimental import pallas as pl
from jax.experimental.pallas import tpu as pltpu
from jax.experimental.pallas import tpu_sc as plsc

_NC = 2
_NS = 16
_NW = _NC * _NS
_LANES = 16

_D = 32
_K = 20
_C = 64
_IDX_CHUNK = 128


def _issue_chunk(co, cemb_hbm, xemb_hbm, cidx, xidx, nidx, crows, xrows,
                 nrows, sem):
    pltpu.async_copy(cemb_hbm.at[cidx.at[pl.ds(co, _C)]], crows, sem)
    pltpu.async_copy(xemb_hbm.at[xidx.at[pl.ds(co, _C)]], xrows, sem)
    for j in range(_C * _K // _IDX_CHUNK):
        pltpu.async_copy(
            xemb_hbm.at[nidx.at[pl.ds(co * _K + j * _IDX_CHUNK, _IDX_CHUNK)]],
            nrows.at[pl.ds(j * _IDX_CHUNK, _IDX_CHUNK)], sem)


def _drain_chunk(co, cemb_hbm, xemb_hbm, cidx, xidx, nidx, crows, xrows,
                 nrows, sem):
    pltpu.make_async_copy(cemb_hbm.at[cidx.at[pl.ds(co, _C)]], crows,
                          sem).wait()
    pltpu.make_async_copy(xemb_hbm.at[xidx.at[pl.ds(co, _C)]], xrows,
                          sem).wait()
    for j in range(_C * _K // _IDX_CHUNK):
        pltpu.make_async_copy(
            xemb_hbm.at[nidx.at[pl.ds(co * _K + j * _IDX_CHUNK, _IDX_CHUNK)]],
            nrows.at[pl.ds(j * _IDX_CHUNK, _IDX_CHUNK)], sem).wait()


def _compute_chunk(lane, crows, xrows, nrows, posb, negb):
    def group(g, _):
        rows = g * _LANES + lane
        nbase = rows * _K

        def pstep(d, acc):
            dcol = jnp.full((_LANES,), d, jnp.int32)
            cvec = plsc.load_gather(crows, [rows, dcol])
            xvec = plsc.load_gather(xrows, [rows, dcol])
            return acc + cvec * xvec

        accp = lax.fori_loop(0, _D, pstep, jnp.zeros((_LANES,), jnp.float32))
        plsc.store_scatter(posb, [rows], accp)

        for h in range(2):
            cregs = [
                plsc.load_gather(
                    crows, [rows, jnp.full((_LANES,), h * 16 + t, jnp.int32)])
                for t in range(16)
            ]

            def kstep(k, _, h=h, cregs=cregs):
                nrow = nbase + k
                parts = []
                for q in range(4):
                    acc = None
                    for t in range(q * 4, q * 4 + 4):
                        dcol = jnp.full((_LANES,), h * 16 + t, jnp.int32)
                        nv = plsc.load_gather(nrows, [nrow, dcol])
                        term = cregs[t] * nv
                        acc = term if acc is None else acc + term
                    parts.append(acc)
                accn = (parts[0] + parts[1]) + (parts[2] + parts[3])
                if h == 0:
                    plsc.store_scatter(negb, [nrow], accn)
                else:
                    plsc.addupdate_scatter(negb, [nrow], accn)
                return 0

            lax.fori_loop(0, _K, kstep, 0)
        return 0

    lax.fori_loop(0, _C // _LANES, group, 0)


def _sc_scores_body(c_hbm, x_hbm, n_hbm, cemb_hbm, xemb_hbm,
                    pos_hbm, negs_hbm,
                    cidx, xidx, nidx, crows0, xrows0, nrows0,
                    crows1, xrows1, nrows1, posb, negb, sem0, sem1,
                    *, pb):
    wid = lax.axis_index("s") * _NC + lax.axis_index("c")
    base = wid * pb

    pltpu.sync_copy(c_hbm.at[pl.ds(base, pb)], cidx)
    pltpu.sync_copy(x_hbm.at[pl.ds(base, pb)], xidx)
    pltpu.sync_copy(n_hbm.at[pl.ds(base * _K, pb * _K)], nidx)

    lane = lax.iota(jnp.int32, _LANES)
    bufs = (
        (crows0, xrows0, nrows0, sem0),
        (crows1, xrows1, nrows1, sem1),
    )
    tbl = (cemb_hbm, xemb_hbm, cidx, xidx, nidx)
    nchunks = pb // _C

    _issue_chunk(0, *tbl, *bufs[0])

    def pair(cc, _):
        co0 = (2 * cc) * _C
        co1 = co0 + _C
        _drain_chunk(co0, *tbl, *bufs[0])
        _issue_chunk(co1, *tbl, *bufs[1])
        _compute_chunk(lane, bufs[0][0], bufs[0][1], bufs[0][2], posb, negb)
        pltpu.sync_copy(posb, pos_hbm.at[pl.ds(base + co0, _C)])
        pltpu.sync_copy(negb, negs_hbm.at[pl.ds((base + co0) * _K, _C * _K)])

        _drain_chunk(co1, *tbl, *bufs[1])

        @pl.when(cc < (nchunks // 2) - 1)
        def _():
            _issue_chunk(co1 + _C, *tbl, *bufs[0])

        _compute_chunk(lane, bufs[1][0], bufs[1][1], bufs[1][2], posb, negb)
        pltpu.sync_copy(posb, pos_hbm.at[pl.ds(base + co1, _C)])
        pltpu.sync_copy(negb, negs_hbm.at[pl.ds((base + co1) * _K, _C * _K)])
        return 0

    lax.fori_loop(0, nchunks // 2, pair, 0)


def _sc_scores(center, context, neg_flat, center_emb, context_emb):
    b = center.shape[0]
    pb = b // _NW
    mesh = plsc.VectorSubcoreMesh(core_axis_name="c", subcore_axis_name="s")
    fn = pl.kernel(
        functools.partial(_sc_scores_body, pb=pb),
        out_type=(
            jax.ShapeDtypeStruct((b,), jnp.float32),
            jax.ShapeDtypeStruct((b * _K,), jnp.float32),
        ),
        mesh=mesh,
        scratch_types=[
            pltpu.VMEM((pb,), jnp.int32),
            pltpu.VMEM((pb,), jnp.int32),
            pltpu.VMEM((pb * _K,), jnp.int32),
            pltpu.VMEM((_C, _D), jnp.float32),
            pltpu.VMEM((_C, _D), jnp.float32),
            pltpu.VMEM((_C * _K, _D), jnp.float32),
            pltpu.VMEM((_C, _D), jnp.float32),
            pltpu.VMEM((_C, _D), jnp.float32),
            pltpu.VMEM((_C * _K, _D), jnp.float32),
            pltpu.VMEM((_C,), jnp.float32),
            pltpu.VMEM((_C * _K,), jnp.float32),
            pltpu.SemaphoreType.DMA,
            pltpu.SemaphoreType.DMA,
        ],
        compiler_params=pltpu.CompilerParams(
            needs_layout_passes=False, use_tc_tiling_on_sc=False),
        name="w2v_sc_scores",
    )
    return fn(center, context, neg_flat, center_emb, context_emb)


def _loss_body(pos_ref, neg_ref, out_ref, *, b, k):
    p = pos_ref[...]
    n = neg_ref[...]
    sp = 1.0 / (1.0 + jnp.exp(-p))
    sn = 1.0 / (1.0 + jnp.exp(-n))
    lp = jnp.log(sp + 1e-9)
    ln = jnp.log(1.0 - sn + 1e-9)
    loss = -(jnp.sum(lp) / b) - (jnp.sum(ln) / (b * k))
    out_ref[...] = jnp.full((1, 1), loss, jnp.float32)


def _tc_loss(pos2d, neg2d, b, k):
    fn = pl.pallas_call(
        functools.partial(_loss_body, b=b, k=k),
        out_shape=jax.ShapeDtypeStruct((1, 1), jnp.float32),
    )
    return fn(pos2d, neg2d)


def kernel(center, context, negative_samples, center_emb, context_emb):
    b = center.shape[0]
    k = negative_samples.shape[1]
    neg_flat = negative_samples.reshape(b * k)
    pos, negs = _sc_scores(center, context, neg_flat, center_emb, context_emb)
    pos2d = pos.reshape(b // 128, 128)
    neg2d = negs.reshape((b * k) // 128, 128)
    loss = _tc_loss(pos2d, neg2d, b, k)
    return loss[0, 0]
